# Initial kernel scaffold; baseline (speedup 1.0000x reference)
import jax
import jax.numpy as jnp
from jax.experimental import pallas as pl


def kernel(x):
    raise NotImplementedError("write your pallas kernel here")


# speedup vs baseline: 1.9449x; 1.9449x over previous
import functools

import jax
import jax.numpy as jnp
import numpy as np
from jax.experimental import pallas as pl
from jax.experimental.pallas import tpu as pltpu


def _interp_matrix_np(n_in: int, n_out: int) -> np.ndarray:
    if n_in == 1:
        return np.ones((n_out, 1), dtype=np.float32)
    src = np.arange(n_out, dtype=np.float64) * (n_in - 1) / (n_out - 1)
    lo = np.clip(np.floor(src).astype(np.int64), 0, n_in - 1)
    hi = np.clip(lo + 1, 0, n_in - 1)
    frac = (src - lo).astype(np.float32)
    m = np.zeros((n_out, n_in), dtype=np.float32)
    m[np.arange(n_out), lo] += 1.0 - frac
    m[np.arange(n_out), hi] += frac
    return m


def _upsample_body(x_ref, ah_ref, awt_ref, o_ref):
    t, h, w = x_ref.shape
    w2 = awt_ref.shape[1]

    xb = x_ref[...].astype(jnp.bfloat16).reshape(t * h, w)
    tmp = jnp.dot(xb, awt_ref[...], preferred_element_type=jnp.float32)
    tmpb = tmp.astype(jnp.bfloat16).reshape(t, h, w2)

    ah = ah_ref[...]
    for i in range(t):
        o_ref[i] = jnp.dot(ah, tmpb[i], preferred_element_type=jnp.float32)


@jax.jit
def _upsample(x: jnp.ndarray) -> jnp.ndarray:
    B, C, H, W = x.shape
    H2, W2 = 2 * H, 2 * W
    N = B * C

    a_h = jnp.asarray(_interp_matrix_np(H, H2), dtype=jnp.bfloat16)
    a_w_t = jnp.asarray(_interp_matrix_np(W, W2).T, dtype=jnp.bfloat16)

    T = 16
    while N % T:
        T //= 2
    x_stacked = x.reshape(N, H, W)

    out = pl.pallas_call(
        _upsample_body,
        out_shape=jax.ShapeDtypeStruct((N, H2, W2), x.dtype),
        grid=(N // T,),
        in_specs=[
            pl.BlockSpec((T, H, W), lambda i: (i, 0, 0)),
            pl.BlockSpec((H2, H), lambda i: (0, 0)),
            pl.BlockSpec((W, W2), lambda i: (0, 0)),
        ],
        out_specs=pl.BlockSpec((T, H2, W2), lambda i: (i, 0, 0)),
        compiler_params=pltpu.CompilerParams(
            dimension_semantics=("parallel",),
            vmem_limit_bytes=64 * 1024 * 1024,
        ),
    )(x_stacked, a_h, a_w_t)

    return out.reshape(B, C, H2, W2)


def kernel(x):
    return _upsample(x)


# speedup vs baseline: 2.2288x; 1.1460x over previous
import functools

import jax
import jax.numpy as jnp
import numpy as np
from jax.experimental import pallas as pl
from jax.experimental.pallas import tpu as pltpu


def _interp_matrix_np(n_in: int, n_out: int) -> np.ndarray:
    if n_in == 1:
        return np.ones((n_out, 1), dtype=np.float32)
    src = np.arange(n_out, dtype=np.float64) * (n_in - 1) / (n_out - 1)
    lo = np.clip(np.floor(src).astype(np.int64), 0, n_in - 1)
    hi = np.clip(lo + 1, 0, n_in - 1)
    frac = (src - lo).astype(np.float32)
    m = np.zeros((n_out, n_in), dtype=np.float32)
    m[np.arange(n_out), lo] += 1.0 - frac
    m[np.arange(n_out), hi] += frac
    return m


def _upsample_body(x_ref, ah_ref, awt_ref, o_ref):
    t, h, w = x_ref.shape
    w2 = awt_ref.shape[1]

    xb = x_ref[...].astype(jnp.bfloat16).reshape(t * h, w)
    tmp = jnp.dot(xb, awt_ref[...], preferred_element_type=jnp.float32)
    tmpb = tmp.astype(jnp.bfloat16).reshape(t, h, w2)

    ah = ah_ref[...]
    for i in range(t):
        o_ref[i] = jnp.dot(ah, tmpb[i], preferred_element_type=jnp.float32)


@jax.jit
def _upsample(x: jnp.ndarray) -> jnp.ndarray:
    B, C, H, W = x.shape
    H2, W2 = 2 * H, 2 * W
    N = B * C

    a_h = jnp.asarray(_interp_matrix_np(H, H2), dtype=jnp.bfloat16)
    a_w_t = jnp.asarray(_interp_matrix_np(W, W2).T, dtype=jnp.bfloat16)

    T = 32
    while N % T:
        T //= 2
    x_stacked = x.reshape(N, H, W)

    out = pl.pallas_call(
        _upsample_body,
        out_shape=jax.ShapeDtypeStruct((N, H2, W2), x.dtype),
        grid=(N // T,),
        in_specs=[
            pl.BlockSpec((T, H, W), lambda i: (i, 0, 0)),
            pl.BlockSpec((H2, H), lambda i: (0, 0)),
            pl.BlockSpec((W, W2), lambda i: (0, 0)),
        ],
        out_specs=pl.BlockSpec((T, H2, W2), lambda i: (i, 0, 0)),
        compiler_params=pltpu.CompilerParams(
            dimension_semantics=("parallel",),
            vmem_limit_bytes=64 * 1024 * 1024,
        ),
    )(x_stacked, a_h, a_w_t)

    return out.reshape(B, C, H2, W2)


def kernel(x):
    return _upsample(x)


# speedup vs baseline: 2.2808x; 1.0233x over previous
import functools

import jax
import jax.numpy as jnp
import numpy as np
from jax.experimental import pallas as pl
from jax.experimental.pallas import tpu as pltpu


def _interp_matrix_np(n_in: int, n_out: int) -> np.ndarray:
    if n_in == 1:
        return np.ones((n_out, 1), dtype=np.float32)
    src = np.arange(n_out, dtype=np.float64) * (n_in - 1) / (n_out - 1)
    lo = np.clip(np.floor(src).astype(np.int64), 0, n_in - 1)
    hi = np.clip(lo + 1, 0, n_in - 1)
    frac = (src - lo).astype(np.float32)
    m = np.zeros((n_out, n_in), dtype=np.float32)
    m[np.arange(n_out), lo] += 1.0 - frac
    m[np.arange(n_out), hi] += frac
    return m


def _upsample_body(x_ref, ah_ref, awt_ref, o_ref):
    t, h, w = x_ref.shape
    w2 = awt_ref.shape[1]

    xb = x_ref[...].astype(jnp.bfloat16).reshape(t * h, w)
    tmp = jnp.dot(xb, awt_ref[...], preferred_element_type=jnp.float32)
    tmpb = tmp.astype(jnp.bfloat16).reshape(t, h, w2)

    ah = ah_ref[...]
    for i in range(t):
        o_ref[i] = jnp.dot(ah, tmpb[i], preferred_element_type=jnp.float32)


@jax.jit
def _upsample(x: jnp.ndarray) -> jnp.ndarray:
    B, C, H, W = x.shape
    H2, W2 = 2 * H, 2 * W
    N = B * C

    a_h = jnp.asarray(_interp_matrix_np(H, H2), dtype=jnp.bfloat16)
    a_w_t = jnp.asarray(_interp_matrix_np(W, W2).T, dtype=jnp.bfloat16)

    T = 64
    while N % T:
        T //= 2
    x_stacked = x.reshape(N, H, W)

    out = pl.pallas_call(
        _upsample_body,
        out_shape=jax.ShapeDtypeStruct((N, H2, W2), x.dtype),
        grid=(N // T,),
        in_specs=[
            pl.BlockSpec((T, H, W), lambda i: (i, 0, 0)),
            pl.BlockSpec((H2, H), lambda i: (0, 0)),
            pl.BlockSpec((W, W2), lambda i: (0, 0)),
        ],
        out_specs=pl.BlockSpec((T, H2, W2), lambda i: (i, 0, 0)),
        compiler_params=pltpu.CompilerParams(
            dimension_semantics=("parallel",),
            vmem_limit_bytes=64 * 1024 * 1024,
        ),
    )(x_stacked, a_h, a_w_t)

    return out.reshape(B, C, H2, W2)


def kernel(x):
    return _upsample(x)
